# baseline (speedup 1.0000x reference)
import functools

import numpy as np

import jax
import jax.numpy as jnp
from jax import lax
from jax.experimental import pallas as pl
from jax.experimental.pallas import tpu as pltpu
from jax.experimental.pallas import tpu_sc as plsc

B = 16384
D = 64
NNEG = 20
NC = 2
NS = 16
L = 16
NW = NC * NS
BPW = B // NW
CHUNK = 64
NCHUNK = BPW // CHUNK
NIDX_BLK = 128
NBLK = CHUNK * NNEG // NIDX_BLK
DK = D // L
RB = 4
NV = RB * NNEG // L

_R0 = [(v * L) // NNEG for v in range(NV)]
_C0 = [(v * L) % NNEG for v in range(NV)]
_BPOS = [NNEG * (r0 + 1) - v * L for v, r0 in enumerate(_R0)]


def _sg_body(t_idx_hbm, c_idx_hbm, n_idx_hbm, t_tab, c_tab,
             pos_hbm, neg_hbm,
             t_idx_v, c_idx_v, n2d_v, n_idx_v, t_rows, c_rows, n_rows,
             pos_v, neg_v, sem):
    wid = lax.axis_index("s") * NC + lax.axis_index("c")
    lane = lax.iota(jnp.int32, L)
    zero16 = lane * 0
    zero16f = zero16.astype(jnp.float32)
    crossed = [lane >= bp for bp in _BPOS]
    rowoff = [jnp.where(crossed[v], _R0[v] + 1, _R0[v]).astype(jnp.int32)
              for v in range(NV)]
    coloff = [lane + jnp.where(crossed[v], _C0[v] - NNEG, _C0[v]
                               ).astype(jnp.int32)
              for v in range(NV)]

    def chunk_body(ch, carry):
        base = wid * BPW + ch * CHUNK
        pltpu.sync_copy(t_idx_hbm.at[pl.ds(base, CHUNK)], t_idx_v)
        pltpu.sync_copy(c_idx_hbm.at[pl.ds(base, CHUNK)], c_idx_v)
        pltpu.sync_copy(n_idx_hbm.at[pl.ds(base, CHUNK)], n2d_v)

        def flat_body(r, fcarry):
            for v in range(NV):
                vals = plsc.load_gather(n2d_v, [r + rowoff[v], coloff[v]])
                n_idx_v[pl.ds(r * NNEG + v * L, L)] = vals
            return fcarry

        lax.fori_loop(0, CHUNK // RB, lambda r, c: flat_body(r * RB, c), 0)

        dmas = [pltpu.async_copy(t_tab.at[t_idx_v], t_rows, sem),
                pltpu.async_copy(c_tab.at[c_idx_v], c_rows, sem)]
        for k in range(NBLK):
            dmas.append(pltpu.async_copy(
                c_tab.at[n_idx_v.at[pl.ds(k * NIDX_BLK, NIDX_BLK)]],
                n_rows.at[pl.ds(k * NIDX_BLK, NIDX_BLK)], sem))
        for dma in dmas:
            dma.wait()

        def group_body(g, gcarry):
            rows = g * L + lane
            rows20 = rows * NNEG
            acc_p = zero16f
            acc_n = [zero16f] * NNEG
            for db in range(D // L):
                dvec = [zero16 + (db * L + dd) for dd in range(L)]
                tv = [plsc.load_gather(t_rows, [rows, dvec[dd]])
                      for dd in range(L)]
                for dd in range(L):
                    cv = plsc.load_gather(c_rows, [rows, dvec[dd]])
                    acc_p = acc_p + tv[dd] * cv
                for j in range(NNEG):
                    rj = rows20 + j
                    a = acc_n[j]
                    for dd in range(L):
                        nv = plsc.load_gather(n_rows, [rj, dvec[dd]])
                        a = a + tv[dd] * nv
                    acc_n[j] = a
            pos_v[pl.ds(g * L, L)] = acc_p
            for j in range(NNEG):
                plsc.store_scatter(neg_v, [rows, zero16 + j], acc_n[j])
            return gcarry

        lax.fori_loop(0, CHUNK // L, group_body, 0)
        pltpu.sync_copy(pos_v, pos_hbm.at[pl.ds(base, CHUNK)])
        pltpu.sync_copy(neg_v, neg_hbm.at[pl.ds(base, CHUNK)])
        return carry

    lax.fori_loop(0, NCHUNK, chunk_body, 0)


_sg_kernel = functools.partial(
    pl.kernel,
    mesh=plsc.VectorSubcoreMesh(core_axis_name="c", subcore_axis_name="s"),
    out_type=[jax.ShapeDtypeStruct((B,), jnp.float32),
              jax.ShapeDtypeStruct((B, NNEG), jnp.float32)],
    scratch_types=[
        pltpu.VMEM((CHUNK,), jnp.int32),
        pltpu.VMEM((CHUNK,), jnp.int32),
        pltpu.VMEM((CHUNK, NNEG), jnp.int32),
        pltpu.VMEM((CHUNK * NNEG,), jnp.int32),
        pltpu.VMEM((CHUNK, D), jnp.float32),
        pltpu.VMEM((CHUNK, D), jnp.float32),
        pltpu.VMEM((CHUNK * NNEG, D), jnp.float32),
        pltpu.VMEM((CHUNK,), jnp.float32),
        pltpu.VMEM((CHUNK, NNEG), jnp.float32),
        pltpu.SemaphoreType.DMA,
    ],
    compiler_params=pltpu.CompilerParams(needs_layout_passes=False,
                                         use_tc_tiling_on_sc=False),
)(_sg_body)


def kernel(target, context, negative_samples, target_table, context_table):
    pos, neg = _sg_kernel(target.astype(jnp.int32),
                          context.astype(jnp.int32),
                          negative_samples.astype(jnp.int32),
                          target_table, context_table)
    return pos, neg


# speedup vs baseline: 1.0172x; 1.0172x over previous
import functools

import jax
import jax.numpy as jnp
from jax import lax
from jax.experimental import pallas as pl
from jax.experimental.pallas import tpu as pltpu
from jax.experimental.pallas import tpu_sc as plsc

VOCAB = 1000000
B = 16384
D = 64
NNEG = 20
NC = 2
NS = 16
L = 16
NW = NC * NS
BPW = B // NW
CHUNK = 32
NCHUNK = BPW // CHUNK
NIDX_BLK = 128
NBLK = CHUNK * NNEG // NIDX_BLK
DK = D // L
W = 2 * D


def _sg_body(t_idx_hbm, c_idx_hbm, n_idx_hbm, t_tab, c_tab,
             pos_hbm, neg_hbm,
             t_idx_v, c_idx_v, n_idx_v, t_pidx_v, c_pidx_v, n_pidx_v,
             t_rows, c_rows, n_rows, pos_v, neg_v, sem):
    wid = lax.axis_index("s") * NC + lax.axis_index("c")
    lane = lax.iota(jnp.int32, L)
    last = lane == (L - 1)
    colb = [lane + k * L for k in range(DK)]

    def chunk_body(ch, carry):
        base = wid * BPW + ch * CHUNK
        pltpu.sync_copy(t_idx_hbm.at[pl.ds(base, CHUNK)], t_idx_v)
        pltpu.sync_copy(c_idx_hbm.at[pl.ds(base, CHUNK)], c_idx_v)
        pltpu.sync_copy(n_idx_hbm.at[pl.ds(base * NNEG, CHUNK * NNEG)],
                        n_idx_v)

        for v in range(CHUNK // L):
            t_pidx_v[pl.ds(v * L, L)] = t_idx_v[pl.ds(v * L, L)] >> 1
            c_pidx_v[pl.ds(v * L, L)] = c_idx_v[pl.ds(v * L, L)] >> 1

        def pair_body(i, pcarry):
            n_pidx_v[pl.ds(i * L, L)] = n_idx_v[pl.ds(i * L, L)] >> 1
            return pcarry

        lax.fori_loop(0, CHUNK * NNEG // L, pair_body, 0)

        dmas = [pltpu.async_copy(t_tab.at[t_pidx_v], t_rows, sem),
                pltpu.async_copy(c_tab.at[c_pidx_v], c_rows, sem)]
        for k in range(NBLK):
            dmas.append(pltpu.async_copy(
                c_tab.at[n_pidx_v.at[pl.ds(k * NIDX_BLK, NIDX_BLK)]],
                n_rows.at[pl.ds(k * NIDX_BLK, NIDX_BLK)], sem))
        for dma in dmas:
            dma.wait()

        def elem_body(e, ecarry):
            fe = jnp.full((L,), e, jnp.int32)
            tpar = (plsc.load_gather(t_idx_v, [fe]) & 1) << 6
            cpar = (plsc.load_gather(c_idx_v, [fe]) & 1) << 6
            tv = [plsc.load_gather(t_rows, [fe, tpar + colb[k]])
                  for k in range(DK)]
            prod = tv[0] * plsc.load_gather(c_rows, [fe, cpar + colb[0]])
            for k in range(1, DK):
                prod = prod + tv[k] * plsc.load_gather(
                    c_rows, [fe, cpar + colb[k]])
            plsc.store_scatter(pos_v, [fe], plsc.cumsum(prod), mask=last)
            for j in range(NNEG):
                r = e * NNEG + j
                fr = jnp.full((L,), r, jnp.int32)
                npar = (plsc.load_gather(n_idx_v, [fr]) & 1) << 6
                prod = tv[0] * plsc.load_gather(n_rows,
                                                [fr, npar + colb[0]])
                for k in range(1, DK):
                    prod = prod + tv[k] * plsc.load_gather(
                        n_rows, [fr, npar + colb[k]])
                plsc.store_scatter(neg_v, [fr], plsc.cumsum(prod),
                                   mask=last)
            return ecarry

        lax.fori_loop(0, CHUNK, elem_body, 0)
        pltpu.sync_copy(pos_v, pos_hbm.at[pl.ds(base, CHUNK)])
        pltpu.sync_copy(neg_v, neg_hbm.at[pl.ds(base * NNEG, CHUNK * NNEG)])
        return carry

    lax.fori_loop(0, NCHUNK, chunk_body, 0)


_sg_kernel = functools.partial(
    pl.kernel,
    mesh=plsc.VectorSubcoreMesh(core_axis_name="c", subcore_axis_name="s"),
    out_type=[jax.ShapeDtypeStruct((B,), jnp.float32),
              jax.ShapeDtypeStruct((B * NNEG,), jnp.float32)],
    scratch_types=[
        pltpu.VMEM((CHUNK,), jnp.int32),
        pltpu.VMEM((CHUNK,), jnp.int32),
        pltpu.VMEM((CHUNK * NNEG,), jnp.int32),
        pltpu.VMEM((CHUNK,), jnp.int32),
        pltpu.VMEM((CHUNK,), jnp.int32),
        pltpu.VMEM((CHUNK * NNEG,), jnp.int32),
        pltpu.VMEM((CHUNK, W), jnp.float32),
        pltpu.VMEM((CHUNK, W), jnp.float32),
        pltpu.VMEM((CHUNK * NNEG, W), jnp.float32),
        pltpu.VMEM((CHUNK,), jnp.float32),
        pltpu.VMEM((CHUNK * NNEG,), jnp.float32),
        pltpu.SemaphoreType.DMA,
    ],
    compiler_params=pltpu.CompilerParams(needs_layout_passes=False),
)(_sg_body)


def kernel(target, context, negative_samples, target_table, context_table):
    pos, neg = _sg_kernel(target.astype(jnp.int32),
                          context.astype(jnp.int32),
                          negative_samples.reshape(-1).astype(jnp.int32),
                          target_table.reshape(VOCAB // 2, W),
                          context_table.reshape(VOCAB // 2, W))
    return pos, neg.reshape(B, NNEG)


# speedup vs baseline: 1.1153x; 1.0964x over previous
import functools

import numpy as np

import jax
import jax.numpy as jnp
from jax import lax
from jax.experimental import pallas as pl
from jax.experimental.pallas import tpu as pltpu
from jax.experimental.pallas import tpu_sc as plsc

B = 16384
D = 64
NNEG = 20
NC = 2
NS = 16
L = 16
NW = NC * NS
BPW = B // NW
CHUNK = 64
NCHUNK = BPW // CHUNK
NIDX_BLK = 128
NBLK = CHUNK * NNEG // NIDX_BLK
DK = D // L
RB = 4
NV = RB * NNEG // L

_R0 = [(v * L) // NNEG for v in range(NV)]
_C0 = [(v * L) % NNEG for v in range(NV)]
_BPOS = [NNEG * (r0 + 1) - v * L for v, r0 in enumerate(_R0)]


def _sg_body(t_idx_hbm, c_idx_hbm, n_idx_hbm, t_tab, c_tab,
             pos_hbm, neg_hbm,
             t_idx_v, c_idx_v, n2d_v, n_idx_v, t_rows, c_rows, n_rows,
             pos_v, neg_v, sem):
    wid = lax.axis_index("s") * NC + lax.axis_index("c")
    lane = lax.iota(jnp.int32, L)
    last = lane == (L - 1)
    crossed = [lane >= bp for bp in _BPOS]
    rowoff = [jnp.where(crossed[v], _R0[v] + 1, _R0[v]).astype(jnp.int32)
              for v in range(NV)]
    coloff = [lane + jnp.where(crossed[v], _C0[v] - NNEG, _C0[v]
                               ).astype(jnp.int32)
              for v in range(NV)]

    def chunk_body(ch, carry):
        base = wid * BPW + ch * CHUNK
        pltpu.sync_copy(t_idx_hbm.at[pl.ds(base, CHUNK)], t_idx_v)
        pltpu.sync_copy(c_idx_hbm.at[pl.ds(base, CHUNK)], c_idx_v)
        pltpu.sync_copy(n_idx_hbm.at[pl.ds(base, CHUNK)], n2d_v)

        def flat_body(r, fcarry):
            for v in range(NV):
                vals = plsc.load_gather(n2d_v, [r + rowoff[v], coloff[v]])
                n_idx_v[pl.ds(r * NNEG + v * L, L)] = vals
            return fcarry

        lax.fori_loop(0, CHUNK // RB, lambda r, c: flat_body(r * RB, c), 0)

        dmas = [pltpu.async_copy(t_tab.at[t_idx_v], t_rows, sem),
                pltpu.async_copy(c_tab.at[c_idx_v], c_rows, sem)]
        for k in range(NBLK):
            dmas.append(pltpu.async_copy(
                c_tab.at[n_idx_v.at[pl.ds(k * NIDX_BLK, NIDX_BLK)]],
                n_rows.at[pl.ds(k * NIDX_BLK, NIDX_BLK)], sem))
        for dma in dmas:
            dma.wait()

        def elem_body(e, ecarry):
            tv = [t_rows[e, pl.ds(k * L, L)] for k in range(DK)]
            prod = tv[0] * c_rows[e, pl.ds(0, L)]
            for k in range(1, DK):
                prod = prod + tv[k] * c_rows[e, pl.ds(k * L, L)]
            plsc.store_scatter(pos_v, [jnp.full((L,), e, jnp.int32)],
                               plsc.cumsum(prod), mask=last)
            for j in range(NNEG):
                r = e * NNEG + j
                prod = tv[0] * n_rows[r, pl.ds(0, L)]
                for k in range(1, DK):
                    prod = prod + tv[k] * n_rows[r, pl.ds(k * L, L)]
                plsc.store_scatter(neg_v,
                                   [jnp.full((L,), e, jnp.int32),
                                    jnp.full((L,), j, jnp.int32)],
                                   plsc.cumsum(prod), mask=last)
            return ecarry

        lax.fori_loop(0, CHUNK, elem_body, 0)
        pltpu.sync_copy(pos_v, pos_hbm.at[pl.ds(base, CHUNK)])
        pltpu.sync_copy(neg_v, neg_hbm.at[pl.ds(base, CHUNK)])
        return carry

    lax.fori_loop(0, NCHUNK, chunk_body, 0)


_sg_kernel = functools.partial(
    pl.kernel,
    mesh=plsc.VectorSubcoreMesh(core_axis_name="c", subcore_axis_name="s"),
    out_type=[jax.ShapeDtypeStruct((B,), jnp.float32),
              jax.ShapeDtypeStruct((B, NNEG), jnp.float32)],
    scratch_types=[
        pltpu.VMEM((CHUNK,), jnp.int32),
        pltpu.VMEM((CHUNK,), jnp.int32),
        pltpu.VMEM((CHUNK, NNEG), jnp.int32),
        pltpu.VMEM((CHUNK * NNEG,), jnp.int32),
        pltpu.VMEM((CHUNK, D), jnp.float32),
        pltpu.VMEM((CHUNK, D), jnp.float32),
        pltpu.VMEM((CHUNK * NNEG, D), jnp.float32),
        pltpu.VMEM((CHUNK,), jnp.float32),
        pltpu.VMEM((CHUNK, NNEG), jnp.float32),
        pltpu.SemaphoreType.DMA,
    ],
    compiler_params=pltpu.CompilerParams(needs_layout_passes=False,
                                         use_tc_tiling_on_sc=False),
)(_sg_body)


def kernel(target, context, negative_samples, target_table, context_table):
    pos, neg = _sg_kernel(target.astype(jnp.int32),
                          context.astype(jnp.int32),
                          negative_samples.astype(jnp.int32),
                          target_table, context_table)
    return pos, neg
